# baseline (speedup 1.0000x reference)
import jax
import jax.numpy as jnp
from jax import lax
from jax.experimental import pallas as pl
from jax.experimental.pallas import tpu as pltpu
from jax.experimental.pallas import tpu_sc as plsc

D = 128
B = 16384
NC = 2
NS = 16
L = 16
NW = NC * NS
B_PER_W = B // NW
CHUNK = 64
NCHUNK = B_PER_W // CHUNK
OB = 5


def _body(w_hbm, o_hbm, i_hbm, res_hbm, idx_v, rows_v, out_v,
          semg, semo, semw):
    wid = lax.axis_index("s") * NC + lax.axis_index("c")
    base = wid * B_PER_W
    pltpu.sync_copy(i_hbm.at[wid], idx_v)

    gathers, outs = [], []
    for j in range(NCHUNK):
        gathers.append(
            pltpu.async_copy(w_hbm.at[idx_v.at[j]], rows_v.at[j], semg.at[j]))
        if j < OB:
            outs.append(
                pltpu.async_copy(o_hbm.at[pl.ds(base + j * CHUNK, CHUNK)],
                                 out_v.at[j], semo.at[j]))
    writes = []
    for j in range(NCHUNK):
        gathers[j].wait()
        outs[j].wait()

        @plsc.parallel_loop(0, CHUNK, unroll=1)
        def mul_row(r):
            for c in range(D // L):
                s = pl.ds(c * L, L)
                rows_v[j, r, s] = rows_v[j, r, s] * out_v[j % OB, r, s]

        writes.append(
            pltpu.async_copy(rows_v.at[j],
                             res_hbm.at[pl.ds(base + j * CHUNK, CHUNK)],
                             semw))
        if j + OB < NCHUNK:
            outs.append(
                pltpu.async_copy(
                    o_hbm.at[pl.ds(base + (j + OB) * CHUNK, CHUNK)],
                    out_v.at[(j + OB) % OB], semo.at[(j + OB) % OB]))
    for w in writes:
        w.wait()


def kernel(out, taget_adj, weight):
    idx = taget_adj.astype(jnp.int32).reshape(NW, NCHUNK, CHUNK)
    mesh = plsc.VectorSubcoreMesh(core_axis_name="c", subcore_axis_name="s")
    k = pl.kernel(
        _body,
        mesh=mesh,
        out_type=jax.ShapeDtypeStruct((B, D), jnp.float32),
        scratch_types=[
            pltpu.VMEM((NCHUNK, CHUNK), jnp.int32),
            pltpu.VMEM((NCHUNK, CHUNK, D), jnp.float32),
            pltpu.VMEM((OB, CHUNK, D), jnp.float32),
            pltpu.SemaphoreType.DMA((NCHUNK,)),
            pltpu.SemaphoreType.DMA((OB,)),
            pltpu.SemaphoreType.DMA,
        ],
    )
    return k(weight, out, idx)


# speedup vs baseline: 1.0420x; 1.0420x over previous
import jax
import jax.numpy as jnp
from jax import lax
from jax.experimental import pallas as pl
from jax.experimental.pallas import tpu as pltpu
from jax.experimental.pallas import tpu_sc as plsc

D = 128
B = 16384
NC = 2
NS = 16
L = 16
NW = NC * NS
B_PER_W = B // NW
CHUNK = 128
NCHUNK = B_PER_W // CHUNK
OB = 3
HALF = CHUNK // 2


def _body(w_hbm, o_hbm, i_hbm, res_hbm, idx_v, rows_v, out_v,
          semg, semo, semw):
    wid = lax.axis_index("s") * NC + lax.axis_index("c")
    base = wid * B_PER_W
    pltpu.sync_copy(i_hbm.at[wid], idx_v)

    gathers, outs = [], []
    for j in range(NCHUNK):
        gathers.append(
            pltpu.async_copy(w_hbm.at[idx_v.at[j]], rows_v.at[j], semg.at[j]))
        if j < OB:
            outs.append(
                pltpu.async_copy(o_hbm.at[pl.ds(base + j * CHUNK, CHUNK)],
                                 out_v.at[j], semo.at[j]))
    writes = []
    for j in range(NCHUNK):
        gathers[j].wait()
        outs[j].wait()

        for h in range(CHUNK // HALF):

            @plsc.parallel_loop(h * HALF, (h + 1) * HALF, unroll=1)
            def mul_row(r):
                for c in range(D // L):
                    s = pl.ds(c * L, L)
                    rows_v[j, r, s] = rows_v[j, r, s] * out_v[j % OB, r, s]

            writes.append(
                pltpu.async_copy(
                    rows_v.at[j, pl.ds(h * HALF, HALF)],
                    res_hbm.at[pl.ds(base + j * CHUNK + h * HALF, HALF)],
                    semw))
        if j + OB < NCHUNK:
            outs.append(
                pltpu.async_copy(
                    o_hbm.at[pl.ds(base + (j + OB) * CHUNK, CHUNK)],
                    out_v.at[(j + OB) % OB], semo.at[(j + OB) % OB]))
    for w in writes:
        w.wait()


def kernel(out, taget_adj, weight):
    idx = taget_adj.astype(jnp.int32).reshape(NW, NCHUNK, CHUNK)
    mesh = plsc.VectorSubcoreMesh(core_axis_name="c", subcore_axis_name="s")
    k = pl.kernel(
        _body,
        mesh=mesh,
        out_type=jax.ShapeDtypeStruct((B, D), jnp.float32),
        scratch_types=[
            pltpu.VMEM((NCHUNK, CHUNK), jnp.int32),
            pltpu.VMEM((NCHUNK, CHUNK, D), jnp.float32),
            pltpu.VMEM((OB, CHUNK, D), jnp.float32),
            pltpu.SemaphoreType.DMA((NCHUNK,)),
            pltpu.SemaphoreType.DMA((OB,)),
            pltpu.SemaphoreType.DMA,
        ],
    )
    return k(weight, out, idx)


# speedup vs baseline: 1.0570x; 1.0143x over previous
import jax
import jax.numpy as jnp
from jax import lax
from jax.experimental import pallas as pl
from jax.experimental.pallas import tpu as pltpu
from jax.experimental.pallas import tpu_sc as plsc

D = 128
B = 16384
NC = 2
NS = 16
L = 16
NW = NC * NS
B_PER_W = B // NW
CHUNK = 128
NCHUNK = B_PER_W // CHUNK
OB = 3


def _body(w_hbm, o_hbm, i_hbm, res_hbm, idx_v, rows_v, out_v,
          semg, semo, semw):
    wid = lax.axis_index("s") * NC + lax.axis_index("c")
    base = wid * B_PER_W
    pltpu.sync_copy(i_hbm.at[pl.ds(base, B_PER_W)], idx_v)

    gathers, outs = [], []
    for j in range(NCHUNK):
        gathers.append(
            pltpu.async_copy(
                w_hbm.at[idx_v.at[pl.ds(j * CHUNK, CHUNK)]],
                rows_v.at[j], semg.at[j]))
        if j < OB:
            outs.append(
                pltpu.async_copy(o_hbm.at[pl.ds(base + j * CHUNK, CHUNK)],
                                 out_v.at[j], semo.at[j]))
    writes = []
    for j in range(NCHUNK):
        gathers[j].wait()
        outs[j].wait()

        @plsc.parallel_loop(0, CHUNK, unroll=1)
        def mul_row(r):
            for c in range(D // L):
                s = pl.ds(c * L, L)
                rows_v[j, r, s] = rows_v[j, r, s] * out_v[j % OB, r, s]

        writes.append(
            pltpu.async_copy(rows_v.at[j],
                             res_hbm.at[pl.ds(base + j * CHUNK, CHUNK)],
                             semw))
        if j + OB < NCHUNK:
            outs.append(
                pltpu.async_copy(
                    o_hbm.at[pl.ds(base + (j + OB) * CHUNK, CHUNK)],
                    out_v.at[(j + OB) % OB], semo.at[(j + OB) % OB]))
    for w in writes:
        w.wait()


def kernel(out, taget_adj, weight):
    idx = taget_adj.astype(jnp.int32)
    mesh = plsc.VectorSubcoreMesh(core_axis_name="c", subcore_axis_name="s")
    k = pl.kernel(
        _body,
        mesh=mesh,
        out_type=jax.ShapeDtypeStruct((B, D), jnp.float32),
        scratch_types=[
            pltpu.VMEM((B_PER_W,), jnp.int32),
            pltpu.VMEM((NCHUNK, CHUNK, D), jnp.float32),
            pltpu.VMEM((OB, CHUNK, D), jnp.float32),
            pltpu.SemaphoreType.DMA((NCHUNK,)),
            pltpu.SemaphoreType.DMA((OB,)),
            pltpu.SemaphoreType.DMA,
        ],
    )
    return k(weight, out, idx)
